# Initial kernel scaffold; baseline (speedup 1.0000x reference)
import jax
import jax.numpy as jnp
from jax.experimental import pallas as pl


def kernel(nodes, children, W1, b1, W2, b2, Wfc, bfc):
    raise NotImplementedError("write your pallas kernel here")


# speedup vs baseline: 2.3379x; 2.3379x over previous
import functools

import jax
import jax.numpy as jnp
from jax import lax
from jax.experimental import pallas as pl
from jax.experimental.pallas import tpu as pltpu
from jax.experimental.pallas import tpu_sc as plsc

_B, _N, _E, _MC = 8, 1024, 128, 16
_BN = _B * _N
_C1, _C2, _LBL = 240, 120, 104
_NW = 32
_NGRP = 8
_NCB = 4
_CB = _E // _NCB
_NODES_W = _BN // _NGRP
_L = 16


def _sc_body(table_hbm, ch_hbm, xr_hbm, xs_hbm, table_v, ch_v, xr_v, xs_v):
    wid = lax.axis_index("s") * 2 + lax.axis_index("c")
    ng = wid // _NCB
    cb = wid % _NCB
    pltpu.sync_copy(table_hbm.at[:, pl.ds(cb * _CB, _CB)], table_v)
    pltpu.sync_copy(ch_hbm.at[:, pl.ds(ng * _NODES_W, _NODES_W)], ch_v)

    def group(g, carry):
        base = g * _L
        one = jnp.full((_L,), 1.0, jnp.float32)
        zero = jnp.zeros((_L,), jnp.float32)
        izero = jnp.zeros((_L,), jnp.int32)
        ns = zero
        for j in range(_MC):
            ns = ns + jnp.where(ch_v[j, pl.ds(base, _L)] != izero, one, zero)
        rec = one / (ns - one)
        is1 = ns == one
        half = jnp.full((_L,), 0.5, jnp.float32)
        for cc in range(_CB // 8):
            accs = [jnp.zeros((_L,), jnp.float32) for _ in range(8)]
            accr = [jnp.zeros((_L,), jnp.float32) for _ in range(8)]
            for j in range(_MC):
                ch = ch_v[j, pl.ds(base, _L)]
                aj = jnp.where(is1, half if j == 0 else zero,
                               jnp.full((_L,), float(j), jnp.float32) * rec)
                for c in range(8):
                    col = jnp.full((_L,), cc * 8 + c, jnp.int32)
                    row = plsc.load_gather(table_v, [ch, col])
                    accs[c] = accs[c] + row
                    accr[c] = accr[c] + aj * row
            for c in range(8):
                xs_v[cc * 8 + c, pl.ds(base, _L)] = accs[c]
                xr_v[cc * 8 + c, pl.ds(base, _L)] = accr[c]
        return carry

    lax.fori_loop(0, _NODES_W // _L, group, 0)
    pltpu.sync_copy(xr_v, xr_hbm.at[pl.ds(cb * _CB, _CB), pl.ds(ng * _NODES_W, _NODES_W)])
    pltpu.sync_copy(xs_v, xs_hbm.at[pl.ds(cb * _CB, _CB), pl.ds(ng * _NODES_W, _NODES_W)])


@jax.jit
def _sc_gather(table, ch_t):
    mesh = plsc.VectorSubcoreMesh(core_axis_name="c", subcore_axis_name="s")
    f = pl.kernel(
        _sc_body,
        out_type=[jax.ShapeDtypeStruct((_E, _BN), jnp.float32),
                  jax.ShapeDtypeStruct((_E, _BN), jnp.float32)],
        mesh=mesh,
        scratch_types=[
            pltpu.VMEM((_N, _CB), jnp.float32),
            pltpu.VMEM((_MC, _NODES_W), jnp.int32),
            pltpu.VMEM((_CB, _NODES_W), jnp.float32),
            pltpu.VMEM((_CB, _NODES_W), jnp.float32),
        ],
        compiler_params=pltpu.CompilerParams(use_tc_tiling_on_sc=False,
                                             needs_layout_passes=False),
    )
    return f(table, ch_t)


def _tc_body(nodes_ref, xr_ref, xs_ref, a0_ref, a1_ref, a2_ref, b1_ref,
             w2_ref, b2_ref, g_ref, bfc_ref, out_ref):
    x = jnp.dot(nodes_ref[...], a0_ref[...], preferred_element_type=jnp.float32)
    x = x + lax.dot_general(xr_ref[...], a1_ref[...], (((0,), (0,)), ((), ())),
                            preferred_element_type=jnp.float32)
    x = x + lax.dot_general(xs_ref[...], a2_ref[...], (((0,), (0,)), ((), ())),
                            preferred_element_type=jnp.float32)
    h1 = jnp.tanh(x + b1_ref[...])
    h2 = jnp.tanh(jnp.dot(h1, w2_ref[...], preferred_element_type=jnp.float32)
                  + b2_ref[...])
    m8 = jnp.max(h2.reshape(8, _N // 8, _C2), axis=1)
    m4 = jnp.max(m8.reshape(4, 2, _C2), axis=1)
    m2 = jnp.max(m4.reshape(2, 2, _C2), axis=1)
    m1 = jnp.max(m2, axis=0, keepdims=True)
    p = jnp.concatenate([m1, m2, m4, m8], axis=0)
    o = jnp.sum(p[:, :, None] * g_ref[...], axis=(0, 1)) + bfc_ref[0]
    out_ref[pl.ds(pl.program_id(0), 1), :] = o[None, :]


@functools.partial(jax.jit, static_argnames=())
def _tc_dense(nodes_f, xr_t, xs_t, a0, a1, a2, b1, w2t, b2, g, bfc):
    full = lambda shape: pl.BlockSpec(shape, lambda b: (0,) * len(shape))
    return pl.pallas_call(
        _tc_body,
        grid=(_B,),
        in_specs=[
            pl.BlockSpec((_N, _E), lambda b: (b, 0)),
            pl.BlockSpec((_E, _N), lambda b: (0, b)),
            pl.BlockSpec((_E, _N), lambda b: (0, b)),
            full((_E, _C1)), full((_E, _C1)), full((_E, _C1)),
            full((1, _C1)), full((_C1, _C2)), full((1, _C2)),
            full((15, _C2, _LBL)), full((1, _LBL)),
        ],
        out_specs=pl.BlockSpec((_B, _LBL), lambda b: (0, 0)),
        out_shape=jax.ShapeDtypeStruct((_B, _LBL), jnp.float32),
    )(nodes_f, xr_t, xs_t, a0, a1, a2, b1, w2t, b2, g, bfc)


def kernel(nodes, children, W1, b1, W2, b2, Wfc, bfc):
    table = jnp.concatenate(
        [jnp.zeros((1, _E), nodes.dtype), nodes[0, 1:, :]], axis=0)
    ch_t = children.reshape(_BN, _MC).T
    xr_t, xs_t = _sc_gather(table, ch_t)

    a0 = W1[:, 0::3].T
    a1 = (W1[:, 1::3] - W1[:, 2::3]).T
    a2 = W1[:, 2::3].T
    g1 = Wfc[:, 0:120].reshape(_LBL, _C2, 1).transpose(2, 1, 0)
    g2 = Wfc[:, 120:360].reshape(_LBL, _C2, 2).transpose(2, 1, 0)
    g3 = Wfc[:, 360:840].reshape(_LBL, _C2, 4).transpose(2, 1, 0)
    g4 = Wfc[:, 840:1800].reshape(_LBL, _C2, 8).transpose(2, 1, 0)
    g = jnp.concatenate([g1, g2, g3, g4], axis=0)
    return _tc_dense(nodes.reshape(_BN, _E), xr_t, xs_t, a0, a1, a2,
                     b1[None, :], W2.T, b2[None, :], g, bfc[None, :])


# speedup vs baseline: 5.8947x; 2.5214x over previous
import functools

import jax
import jax.numpy as jnp
from jax import lax
from jax.experimental import pallas as pl
from jax.experimental.pallas import tpu as pltpu
from jax.experimental.pallas import tpu_sc as plsc

_B, _N, _E, _MC = 8, 1024, 128, 16
_BN = _B * _N
_C1, _C2, _LBL = 240, 120, 104
_NW = 32
_NGRP = 8
_NCB = 4
_CB = _E // _NCB
_NODES_W = _BN // _NGRP
_L = 16


def _sc_body(table_hbm, ch_hbm, xr_hbm, xs_hbm, table_v, ch_v, xr_v, xs_v):
    wid = lax.axis_index("s") * 2 + lax.axis_index("c")
    ng = wid // _NCB
    cb = wid % _NCB
    pltpu.sync_copy(table_hbm.at[:, pl.ds(cb * _CB, _CB)],
                    table_v.at[:, pl.ds(0, _CB)])
    pltpu.sync_copy(ch_hbm.at[:, pl.ds(ng * _NODES_W, _NODES_W)], ch_v)

    def group(g, carry):
        base = g * _L
        one = jnp.full((_L,), 1.0, jnp.float32)
        zero = jnp.zeros((_L,), jnp.float32)
        izero = jnp.zeros((_L,), jnp.int32)
        ns = zero
        for j in range(_MC):
            ns = ns + jnp.where(ch_v[j, pl.ds(base, _L)] != izero, one, zero)
        rec = one / (ns - one)
        is1 = ns == one
        half = jnp.full((_L,), 0.5, jnp.float32)
        for cc in range(_CB // 8):
            accs = [jnp.zeros((_L,), jnp.float32) for _ in range(8)]
            accr = [jnp.zeros((_L,), jnp.float32) for _ in range(8)]
            for j in range(_MC):
                ch = ch_v[j, pl.ds(base, _L)]
                aj = jnp.where(is1, half if j == 0 else zero,
                               jnp.full((_L,), float(j), jnp.float32) * rec)
                for c in range(8):
                    col = jnp.full((_L,), cc * 8 + c, jnp.int32)
                    row = plsc.load_gather(table_v, [ch, col])
                    accs[c] = accs[c] + row
                    accr[c] = accr[c] + aj * row
            for c in range(8):
                xs_v[cc * 8 + c, pl.ds(base, _L)] = accs[c]
                xr_v[cc * 8 + c, pl.ds(base, _L)] = accr[c]
        return carry

    lax.fori_loop(0, _NODES_W // _L, group, 0)
    pltpu.sync_copy(xr_v, xr_hbm.at[pl.ds(cb * _CB, _CB), pl.ds(ng * _NODES_W, _NODES_W)])
    pltpu.sync_copy(xs_v, xs_hbm.at[pl.ds(cb * _CB, _CB), pl.ds(ng * _NODES_W, _NODES_W)])


@jax.jit
def _sc_gather(table, ch_t):
    mesh = plsc.VectorSubcoreMesh(core_axis_name="c", subcore_axis_name="s")
    f = pl.kernel(
        _sc_body,
        out_type=[jax.ShapeDtypeStruct((_E, _BN), jnp.float32),
                  jax.ShapeDtypeStruct((_E, _BN), jnp.float32)],
        mesh=mesh,
        scratch_types=[
            pltpu.VMEM((_N, _CB + 1), jnp.float32),
            pltpu.VMEM((_MC, _NODES_W), jnp.int32),
            pltpu.VMEM((_CB, _NODES_W), jnp.float32),
            pltpu.VMEM((_CB, _NODES_W), jnp.float32),
        ],
        compiler_params=pltpu.CompilerParams(use_tc_tiling_on_sc=False,
                                             needs_layout_passes=False),
    )
    return f(table, ch_t)


def _tc_body(nodes_ref, xr_ref, xs_ref, a0_ref, a1_ref, a2_ref, b1_ref,
             w2_ref, b2_ref, g_ref, bfc_ref, out_ref):
    x = jnp.dot(nodes_ref[...], a0_ref[...], preferred_element_type=jnp.float32)
    x = x + lax.dot_general(xr_ref[...], a1_ref[...], (((0,), (0,)), ((), ())),
                            preferred_element_type=jnp.float32)
    x = x + lax.dot_general(xs_ref[...], a2_ref[...], (((0,), (0,)), ((), ())),
                            preferred_element_type=jnp.float32)
    h1 = jnp.tanh(x + b1_ref[...])
    h2 = jnp.tanh(jnp.dot(h1, w2_ref[...], preferred_element_type=jnp.float32)
                  + b2_ref[...])
    m8 = jnp.max(h2.reshape(8, _N // 8, _C2), axis=1)
    m4 = jnp.max(m8.reshape(4, 2, _C2), axis=1)
    m2 = jnp.max(m4.reshape(2, 2, _C2), axis=1)
    m1 = jnp.max(m2, axis=0, keepdims=True)
    p = jnp.concatenate([m1, m2, m4, m8], axis=0)
    o = jnp.sum(p[:, :, None] * g_ref[...], axis=(0, 1)) + bfc_ref[0]
    out_ref[pl.ds(pl.program_id(0), 1), :] = o[None, :]


@functools.partial(jax.jit, static_argnames=())
def _tc_dense(nodes_f, xr_t, xs_t, a0, a1, a2, b1, w2t, b2, g, bfc):
    full = lambda shape: pl.BlockSpec(shape, lambda b: (0,) * len(shape))
    return pl.pallas_call(
        _tc_body,
        grid=(_B,),
        in_specs=[
            pl.BlockSpec((_N, _E), lambda b: (b, 0)),
            pl.BlockSpec((_E, _N), lambda b: (0, b)),
            pl.BlockSpec((_E, _N), lambda b: (0, b)),
            full((_E, _C1)), full((_E, _C1)), full((_E, _C1)),
            full((1, _C1)), full((_C1, _C2)), full((1, _C2)),
            full((15, _C2, _LBL)), full((1, _LBL)),
        ],
        out_specs=pl.BlockSpec((_B, _LBL), lambda b: (0, 0)),
        out_shape=jax.ShapeDtypeStruct((_B, _LBL), jnp.float32),
    )(nodes_f, xr_t, xs_t, a0, a1, a2, b1, w2t, b2, g, bfc)


def kernel(nodes, children, W1, b1, W2, b2, Wfc, bfc):
    table = jnp.concatenate(
        [jnp.zeros((1, _E), nodes.dtype), nodes[0, 1:, :]], axis=0)
    ch_t = children.reshape(_BN, _MC).T
    xr_t, xs_t = _sc_gather(table, ch_t)

    a0 = W1[:, 0::3].T
    a1 = (W1[:, 1::3] - W1[:, 2::3]).T
    a2 = W1[:, 2::3].T
    g1 = Wfc[:, 0:120].reshape(_LBL, _C2, 1).transpose(2, 1, 0)
    g2 = Wfc[:, 120:360].reshape(_LBL, _C2, 2).transpose(2, 1, 0)
    g3 = Wfc[:, 360:840].reshape(_LBL, _C2, 4).transpose(2, 1, 0)
    g4 = Wfc[:, 840:1800].reshape(_LBL, _C2, 8).transpose(2, 1, 0)
    g = jnp.concatenate([g1, g2, g3, g4], axis=0)
    return _tc_dense(nodes.reshape(_BN, _E), xr_t, xs_t, a0, a1, a2,
                     b1[None, :], W2.T, b2[None, :], g, bfc[None, :])


# speedup vs baseline: 8.5250x; 1.4462x over previous
import functools

import jax
import jax.numpy as jnp
from jax import lax
from jax.experimental import pallas as pl
from jax.experimental.pallas import tpu as pltpu
from jax.experimental.pallas import tpu_sc as plsc

_B, _N, _E, _MC = 8, 1024, 128, 16
_BN = _B * _N
_C1, _C2, _LBL = 240, 120, 104
_NW = 32
_NGRP = 8
_NCB = 4
_CB = _E // _NCB
_NODES_W = _BN // _NGRP
_L = 16


def _sc_body(table_hbm, ch_hbm, xr_hbm, xs_hbm, table_v, ch_v, xr_v, xs_v):
    wid = lax.axis_index("s") * 2 + lax.axis_index("c")
    ng = wid // _NCB
    cb = wid % _NCB
    pltpu.sync_copy(table_hbm.at[:, pl.ds(cb * _CB, _CB)], table_v)
    pltpu.sync_copy(ch_hbm.at[:, pl.ds(ng * _NODES_W, _NODES_W)], ch_v)

    def group(g, carry):
        base = g * _L
        one = jnp.full((_L,), 1.0, jnp.float32)
        zero = jnp.zeros((_L,), jnp.float32)
        izero = jnp.zeros((_L,), jnp.int32)
        cvs = [ch_v[j, pl.ds(base, _L)] for j in range(_MC)]
        ns = zero
        for j in range(_MC):
            ns = ns + jnp.where(cvs[j] != izero, one, zero)
        recv = one / (ns - one)
        m1v = jnp.where(ns == one, one, zero)
        for n in range(_L):
            node = base + n
            rec = recv[n]
            m1 = m1v[n]
            ch = cvs[_MC - 1][n]
            run0 = table_v[ch, pl.ds(0, _L)]
            run1 = table_v[ch, pl.ds(_L, _L)]
            xj0 = run0
            xj1 = run1
            for j in range(_MC - 2, 0, -1):
                ch = cvs[j][n]
                run0 = run0 + table_v[ch, pl.ds(0, _L)]
                run1 = run1 + table_v[ch, pl.ds(_L, _L)]
                xj0 = xj0 + run0
                xj1 = xj1 + run1
            ch = cvs[0][n]
            r00 = table_v[ch, pl.ds(0, _L)]
            r01 = table_v[ch, pl.ds(_L, _L)]
            run0 = run0 + r00
            run1 = run1 + r01
            cond = jnp.broadcast_to(m1, (_L,)) > 0.5
            xr0 = jnp.where(cond, 0.5 * r00, rec * xj0)
            xr1 = jnp.where(cond, 0.5 * r01, rec * xj1)
            xs_v[node, pl.ds(0, _L)] = run0
            xs_v[node, pl.ds(_L, _L)] = run1
            xr_v[node, pl.ds(0, _L)] = xr0
            xr_v[node, pl.ds(_L, _L)] = xr1
        return carry

    lax.fori_loop(0, _NODES_W // _L, group, 0)
    pltpu.sync_copy(xr_v, xr_hbm.at[pl.ds(ng * _NODES_W, _NODES_W), pl.ds(cb * _CB, _CB)])
    pltpu.sync_copy(xs_v, xs_hbm.at[pl.ds(ng * _NODES_W, _NODES_W), pl.ds(cb * _CB, _CB)])


@jax.jit
def _sc_gather(table, ch_t):
    mesh = plsc.VectorSubcoreMesh(core_axis_name="c", subcore_axis_name="s")
    f = pl.kernel(
        _sc_body,
        out_type=[jax.ShapeDtypeStruct((_BN, _E), jnp.float32),
                  jax.ShapeDtypeStruct((_BN, _E), jnp.float32)],
        mesh=mesh,
        scratch_types=[
            pltpu.VMEM((_N, _CB), jnp.float32),
            pltpu.VMEM((_MC, _NODES_W), jnp.int32),
            pltpu.VMEM((_NODES_W, _CB), jnp.float32),
            pltpu.VMEM((_NODES_W, _CB), jnp.float32),
        ],
        compiler_params=pltpu.CompilerParams(use_tc_tiling_on_sc=False,
                                             needs_layout_passes=False),
    )
    return f(table, ch_t)


def _tc_body(nodes_ref, xr_ref, xs_ref, a0_ref, a1_ref, a2_ref, b1_ref,
             w2_ref, b2_ref, g_ref, bfc_ref, out_ref):
    x = jnp.dot(nodes_ref[...], a0_ref[...], preferred_element_type=jnp.float32)
    x = x + jnp.dot(xr_ref[...], a1_ref[...], preferred_element_type=jnp.float32)
    x = x + jnp.dot(xs_ref[...], a2_ref[...], preferred_element_type=jnp.float32)
    h1 = jnp.tanh(x + b1_ref[...])
    h2 = jnp.tanh(jnp.dot(h1, w2_ref[...], preferred_element_type=jnp.float32)
                  + b2_ref[...])
    m8 = jnp.max(h2.reshape(8, _N // 8, _C2), axis=1)
    m4 = jnp.max(m8.reshape(4, 2, _C2), axis=1)
    m2 = jnp.max(m4.reshape(2, 2, _C2), axis=1)
    m1 = jnp.max(m2, axis=0, keepdims=True)
    p = jnp.concatenate([m1, m2, m4, m8], axis=0)
    o = jnp.sum(p[:, :, None] * g_ref[...], axis=(0, 1)) + bfc_ref[0]
    out_ref[pl.ds(pl.program_id(0), 1), :] = o[None, :]


@functools.partial(jax.jit, static_argnames=())
def _tc_dense(nodes_f, xr_t, xs_t, a0, a1, a2, b1, w2t, b2, g, bfc):
    full = lambda shape: pl.BlockSpec(shape, lambda b: (0,) * len(shape))
    return pl.pallas_call(
        _tc_body,
        grid=(_B,),
        in_specs=[
            pl.BlockSpec((_N, _E), lambda b: (b, 0)),
            pl.BlockSpec((_N, _E), lambda b: (b, 0)),
            pl.BlockSpec((_N, _E), lambda b: (b, 0)),
            full((_E, _C1)), full((_E, _C1)), full((_E, _C1)),
            full((1, _C1)), full((_C1, _C2)), full((1, _C2)),
            full((15, _C2, _LBL)), full((1, _LBL)),
        ],
        out_specs=pl.BlockSpec((_B, _LBL), lambda b: (0, 0)),
        out_shape=jax.ShapeDtypeStruct((_B, _LBL), jnp.float32),
    )(nodes_f, xr_t, xs_t, a0, a1, a2, b1, w2t, b2, g, bfc)


def kernel(nodes, children, W1, b1, W2, b2, Wfc, bfc):
    table = jnp.concatenate(
        [jnp.zeros((1, _E), nodes.dtype), nodes[0, 1:, :]], axis=0)
    ch_t = children.reshape(_BN, _MC).T
    xr_t, xs_t = _sc_gather(table, ch_t)

    a0 = W1[:, 0::3].T
    a1 = (W1[:, 1::3] - W1[:, 2::3]).T
    a2 = W1[:, 2::3].T
    g1 = Wfc[:, 0:120].reshape(_LBL, _C2, 1).transpose(2, 1, 0)
    g2 = Wfc[:, 120:360].reshape(_LBL, _C2, 2).transpose(2, 1, 0)
    g3 = Wfc[:, 360:840].reshape(_LBL, _C2, 4).transpose(2, 1, 0)
    g4 = Wfc[:, 840:1800].reshape(_LBL, _C2, 8).transpose(2, 1, 0)
    g = jnp.concatenate([g1, g2, g3, g4], axis=0)
    return _tc_dense(nodes.reshape(_BN, _E), xr_t, xs_t, a0, a1, a2,
                     b1[None, :], W2.T, b2[None, :], g, bfc[None, :])
